# baseline (speedup 1.0000x reference)
import functools

import jax
import jax.numpy as jnp
from jax import lax
from jax.experimental import pallas as pl
from jax.experimental.pallas import tpu as pltpu
from jax.experimental.pallas import tpu_sc as plsc

ROWS = 16384
COLS = 200
NSEL = 100

_info = plsc.get_sparse_core_info()
NC, NS, L = _info.num_cores, _info.num_subcores, _info.num_lanes
NW = NC * NS
KSC = 32
KTC = NSEL - KSC
NBLK = KSC // L
SPLIT = 32
SEG = ROWS // SPLIT
TASKS_PER_W = NBLK * SPLIT // NW

_mesh = plsc.VectorSubcoreMesh(core_axis_name="c", subcore_axis_name="s")


@functools.partial(
    pl.kernel,
    mesh=_mesh,
    out_type=jax.ShapeDtypeStruct((KSC, ROWS), jnp.float32),
    scratch_types=[
        pltpu.VMEM((NSEL,), jnp.int32),
        pltpu.VMEM((L, SEG), jnp.float32),
        pltpu.VMEM((L, SEG), jnp.float32),
        pltpu.SemaphoreType.DMA,
        pltpu.SemaphoreType.DMA,
        pltpu.SemaphoreType.DMA,
        pltpu.SemaphoreType.DMA,
    ],
    compiler_params=pltpu.CompilerParams(
        needs_layout_passes=False, use_tc_tiling_on_sc=True
    ),
)
def _sc_rowgather(inT_hbm, idx_hbm, outT_hbm, idx_v, b0, b1, g0, g1, s0, s1):
    wid = lax.axis_index("s") * NC + lax.axis_index("c")
    pltpu.sync_copy(idx_hbm, idx_v)
    lanes = lax.iota(jnp.int32, L)
    bufs = (b0, b1)
    gsems = (g0, g1)
    ssems = (s0, s1)

    def coords(k):
        t = wid * TASKS_PER_W + k
        j0 = (t % NBLK) * L
        col0 = pl.multiple_of((t // NBLK) * SEG, SEG)
        return j0, col0

    def start_gather(k):
        j0, col0 = coords(k)
        rows = plsc.load_gather(idx_v, [j0 + lanes])
        cp = pltpu.make_async_copy(
            inT_hbm.at[rows, pl.ds(col0, SEG)], bufs[k % 2], gsems[k % 2]
        )
        cp.start()
        return cp

    def start_scatter(k):
        j0, col0 = coords(k)
        cp = pltpu.make_async_copy(
            bufs[k % 2], outT_hbm.at[pl.ds(j0, L), pl.ds(col0, SEG)],
            ssems[k % 2],
        )
        cp.start()
        return cp

    gd = [start_gather(k) for k in range(TASKS_PER_W)]
    pend = []
    for k in range(TASKS_PER_W):
        gd[k].wait()
        pend.append(start_scatter(k))
    for cp in pend:
        cp.wait()


def _tc_body(idx_ref, in_ref, out_ref):
    out_ref[...] = in_ref[...]


_tc_rowgather = pl.pallas_call(
    _tc_body,
    grid_spec=pltpu.PrefetchScalarGridSpec(
        num_scalar_prefetch=1,
        grid=(KTC,),
        in_specs=[
            pl.BlockSpec(
                (1, 1, ROWS), lambda i, idx_ref: (idx_ref[KSC + i], 0, 0)
            ),
        ],
        out_specs=pl.BlockSpec((1, 1, ROWS), lambda i, idx_ref: (i, 0, 0)),
    ),
    out_shape=jax.ShapeDtypeStruct((KTC, 1, ROWS), jnp.float32),
)


def kernel(inputs, categorical_idx):
    xT = inputs.T
    sc_part = _sc_rowgather(xT, categorical_idx)
    tc_part = _tc_rowgather(categorical_idx, xT.reshape(COLS, 1, ROWS))
    outT = jnp.concatenate([sc_part, tc_part.reshape(KTC, ROWS)], axis=0)
    return outT.T


# speedup vs baseline: 2.6952x; 2.6952x over previous
import functools

import jax
import jax.numpy as jnp
from jax import lax
from jax.experimental import pallas as pl
from jax.experimental.pallas import tpu as pltpu
from jax.experimental.pallas import tpu_sc as plsc

ROWS = 16384
COLS = 200
NSEL = 100

_info = plsc.get_sparse_core_info()
NC, NS, L = _info.num_cores, _info.num_subcores, _info.num_lanes
NW = NC * NS
NBLK = NSEL // L
NTAIL = NSEL - NBLK * L
SPLIT = 16
SEG = ROWS // SPLIT
SEGT = ROWS // NW
FULL_PER_W = NBLK * SPLIT // NW

_mesh = plsc.VectorSubcoreMesh(core_axis_name="c", subcore_axis_name="s")


@functools.partial(
    pl.kernel,
    mesh=_mesh,
    out_type=jax.ShapeDtypeStruct((NSEL, ROWS), jnp.float32),
    scratch_types=[
        pltpu.VMEM((NSEL,), jnp.int32),
        pltpu.VMEM((L, SEG), jnp.float32),
        pltpu.VMEM((L, SEG), jnp.float32),
        pltpu.VMEM((L, SEG), jnp.float32),
        pltpu.VMEM((L, SEG), jnp.float32),
        pltpu.SemaphoreType.DMA,
        pltpu.SemaphoreType.DMA,
        pltpu.SemaphoreType.DMA,
        pltpu.SemaphoreType.DMA,
        pltpu.SemaphoreType.DMA,
        pltpu.SemaphoreType.DMA,
        pltpu.SemaphoreType.DMA,
        pltpu.SemaphoreType.DMA,
    ],
    compiler_params=pltpu.CompilerParams(
        needs_layout_passes=False, use_tc_tiling_on_sc=True
    ),
)
def _sc_rowgather(inT_hbm, idx_hbm, outT_hbm, idx_v,
                  b0, b1, b2, b3, g0, g1, g2, g3, s0, s1, s2, s3):
    wid = lax.axis_index("s") * NC + lax.axis_index("c")
    pltpu.sync_copy(idx_hbm, idx_v)
    lanes = lax.iota(jnp.int32, L)
    bufs = (b0, b1, b2, b3)
    gsems = (g0, g1, g2, g3)
    ssems = (s0, s1, s2, s3)
    NT = FULL_PER_W + 1

    def coords(k):
        if k < FULL_PER_W:
            t = wid * FULL_PER_W + k
            j0 = (t % NBLK) * L
            col0 = pl.multiple_of((t // NBLK) * SEG, SEG)
        else:
            j0 = NBLK * L
            col0 = pl.multiple_of(wid * SEGT, SEGT)
        return j0, col0

    def start_gather(k):
        j0, col0 = coords(k)
        rows = plsc.load_gather(idx_v, [jnp.minimum(j0 + lanes, NSEL - 1)])
        if k < FULL_PER_W:
            src = inT_hbm.at[rows, pl.ds(col0, SEG)]
            dst = bufs[k % 4]
        else:
            src = inT_hbm.at[rows, pl.ds(col0, SEGT)]
            dst = bufs[k % 4].at[:, pl.ds(0, SEGT)]
        cp = pltpu.make_async_copy(src, dst, gsems[k % 4])
        cp.start()
        return cp

    def start_scatter(k):
        j0, col0 = coords(k)
        b = k % 4
        if k < FULL_PER_W:
            cp = pltpu.make_async_copy(
                bufs[b], outT_hbm.at[pl.ds(j0, L), pl.ds(col0, SEG)], ssems[b]
            )
            cp.start()
            return [cp]
        descs = []
        for i in range(NTAIL):
            cp = pltpu.make_async_copy(
                bufs[b].at[i, pl.ds(0, SEGT)],
                outT_hbm.at[j0 + i, pl.ds(col0, SEGT)],
                ssems[b],
            )
            cp.start()
            descs.append(cp)
        return descs

    gd = {0: start_gather(0), 1: start_gather(1)}
    pend = [None, None, None, None]
    for k in range(NT):
        gd[k].wait()
        pend[k % 4] = start_scatter(k)
        nk = k + 2
        if nk < NT:
            nb = nk % 4
            if pend[nb] is not None:
                for cp in pend[nb]:
                    cp.wait()
                pend[nb] = None
            gd[nk] = start_gather(nk)
    for ds_list in pend:
        if ds_list is not None:
            for cp in ds_list:
                cp.wait()


def kernel(inputs, categorical_idx):
    outT = _sc_rowgather(inputs.T, categorical_idx)
    return outT.T


# speedup vs baseline: 2.8152x; 1.0445x over previous
import functools

import jax
import jax.numpy as jnp
from jax import lax
from jax.experimental import pallas as pl
from jax.experimental.pallas import tpu as pltpu
from jax.experimental.pallas import tpu_sc as plsc

ROWS = 16384
COLS = 200
NSEL = 100

_info = plsc.get_sparse_core_info()
NC, NS, L = _info.num_cores, _info.num_subcores, _info.num_lanes
NW = NC * NS
NBLK = NSEL // L
NTAIL = NSEL - NBLK * L
SPLIT = 16
SEG = ROWS // SPLIT
SEGT = ROWS // NW
FULL_PER_W = NBLK * SPLIT // NW

_mesh = plsc.VectorSubcoreMesh(core_axis_name="c", subcore_axis_name="s")


@functools.partial(
    pl.kernel,
    mesh=_mesh,
    out_type=jax.ShapeDtypeStruct((NSEL, ROWS), jnp.float32),
    scratch_types=[
        pltpu.VMEM((NSEL,), jnp.int32),
        pltpu.VMEM((L, SEG), jnp.float32),
        pltpu.VMEM((L, SEG), jnp.float32),
        pltpu.VMEM((L, SEG), jnp.float32),
        pltpu.VMEM((L, SEG), jnp.float32),
        pltpu.SemaphoreType.DMA,
        pltpu.SemaphoreType.DMA,
        pltpu.SemaphoreType.DMA,
        pltpu.SemaphoreType.DMA,
        pltpu.SemaphoreType.DMA,
        pltpu.SemaphoreType.DMA,
        pltpu.SemaphoreType.DMA,
        pltpu.SemaphoreType.DMA,
    ],
    compiler_params=pltpu.CompilerParams(
        needs_layout_passes=False, use_tc_tiling_on_sc=True
    ),
)
def _sc_rowgather(inT_hbm, idx_hbm, outT_hbm, idx_v,
                  b0, b1, b2, b3, g0, g1, g2, g3, s0, s1, s2, s3):
    wid = lax.axis_index("s") * NC + lax.axis_index("c")
    pltpu.sync_copy(idx_hbm, idx_v)
    lanes = lax.iota(jnp.int32, L)
    bufs = (b0, b1, b2, b3)
    gsems = (g0, g1, g2, g3)
    ssems = (s0, s1, s2, s3)
    NT = FULL_PER_W + 1

    def coords(k):
        if k < FULL_PER_W:
            t = wid * FULL_PER_W + k
            j0 = (t % NBLK) * L
            col0 = pl.multiple_of((t // NBLK) * SEG, SEG)
        else:
            j0 = NBLK * L
            col0 = pl.multiple_of(wid * SEGT, SEGT)
        return j0, col0

    def start_gather(k):
        j0, col0 = coords(k)
        rows = plsc.load_gather(idx_v, [jnp.minimum(j0 + lanes, NSEL - 1)])
        if k < FULL_PER_W:
            src = inT_hbm.at[rows, pl.ds(col0, SEG)]
            dst = bufs[k % 4]
        else:
            src = inT_hbm.at[rows, pl.ds(col0, SEGT)]
            dst = bufs[k % 4].at[:, pl.ds(0, SEGT)]
        cp = pltpu.make_async_copy(src, dst, gsems[k % 4])
        cp.start()
        return cp

    def start_scatter(k):
        j0, col0 = coords(k)
        b = k % 4
        if k < FULL_PER_W:
            cp = pltpu.make_async_copy(
                bufs[b], outT_hbm.at[pl.ds(j0, L), pl.ds(col0, SEG)], ssems[b]
            )
            cp.start()
            return [cp]
        descs = []
        for i in range(NTAIL):
            cp = pltpu.make_async_copy(
                bufs[b].at[i, pl.ds(0, SEGT)],
                outT_hbm.at[j0 + i, pl.ds(col0, SEGT)],
                ssems[b],
            )
            cp.start()
            descs.append(cp)
        return descs

    gd = [start_gather(k) for k in range(NT)]
    pend = []
    for k in range(NT):
        gd[k].wait()
        pend.extend(start_scatter(k))
    for cp in pend:
        cp.wait()


def kernel(inputs, categorical_idx):
    outT = _sc_rowgather(inputs.T, categorical_idx)
    return outT.T


# speedup vs baseline: 2.8230x; 1.0028x over previous
import functools

import jax
import jax.numpy as jnp
from jax import lax
from jax.experimental import pallas as pl
from jax.experimental.pallas import tpu as pltpu
from jax.experimental.pallas import tpu_sc as plsc

ROWS = 16384
COLS = 200
NSEL = 100

_info = plsc.get_sparse_core_info()
NC, NS, L = _info.num_cores, _info.num_subcores, _info.num_lanes
NW = NC * NS
NBLK = NSEL // L
NTAIL = NSEL - NBLK * L
SPLIT = 16
SEG = ROWS // SPLIT
SEGT = ROWS // NW
FULL_PER_W = NBLK * SPLIT // NW

_mesh = plsc.VectorSubcoreMesh(core_axis_name="c", subcore_axis_name="s")


@functools.partial(
    pl.kernel,
    mesh=_mesh,
    out_type=jax.ShapeDtypeStruct((NSEL, ROWS), jnp.float32),
    scratch_types=[
        pltpu.VMEM((NSEL,), jnp.int32),
        pltpu.VMEM((L, SEG), jnp.float32),
        pltpu.VMEM((L, SEG), jnp.float32),
        pltpu.VMEM((L, SEG), jnp.float32),
        pltpu.VMEM((L, SEG), jnp.float32),
        pltpu.SemaphoreType.DMA,
        pltpu.SemaphoreType.DMA,
        pltpu.SemaphoreType.DMA,
        pltpu.SemaphoreType.DMA,
        pltpu.SemaphoreType.DMA,
        pltpu.SemaphoreType.DMA,
        pltpu.SemaphoreType.DMA,
        pltpu.SemaphoreType.DMA,
    ],
    compiler_params=pltpu.CompilerParams(
        needs_layout_passes=False, use_tc_tiling_on_sc=True
    ),
)
def _sc_rowgather(inT_hbm, idx_hbm, outT_hbm, idx_v,
                  b0, b1, b2, b3, g0, g1, g2, g3, s0, s1, s2, s3):
    wid = lax.axis_index("s") * NC + lax.axis_index("c")
    pltpu.sync_copy(idx_hbm, idx_v)
    lanes = lax.iota(jnp.int32, L)
    bufs = (b0, b1, b2, b3)
    gsems = (g0, g1, g2, g3)
    ssems = (s0, s1, s2, s3)
    NT = FULL_PER_W + 1

    def coords(k):
        if k >= 1:
            t = wid * FULL_PER_W + (k - 1)
            j0 = (t % NBLK) * L
            col0 = pl.multiple_of((t // NBLK) * SEG, SEG)
        else:
            j0 = NBLK * L
            col0 = pl.multiple_of(wid * SEGT, SEGT)
        return j0, col0

    def start_gather(k):
        j0, col0 = coords(k)
        rows = plsc.load_gather(idx_v, [jnp.minimum(j0 + lanes, NSEL - 1)])
        if k >= 1:
            src = inT_hbm.at[rows, pl.ds(col0, SEG)]
            dst = bufs[k % 4]
        else:
            src = inT_hbm.at[rows, pl.ds(col0, SEGT)]
            dst = bufs[k % 4].at[:, pl.ds(0, SEGT)]
        cp = pltpu.make_async_copy(src, dst, gsems[k % 4])
        cp.start()
        return cp

    def start_scatter(k):
        j0, col0 = coords(k)
        b = k % 4
        if k >= 1:
            cp = pltpu.make_async_copy(
                bufs[b], outT_hbm.at[pl.ds(j0, L), pl.ds(col0, SEG)], ssems[b]
            )
            cp.start()
            return [cp]
        descs = []
        for i in range(NTAIL):
            cp = pltpu.make_async_copy(
                bufs[b].at[i, pl.ds(0, SEGT)],
                outT_hbm.at[j0 + i, pl.ds(col0, SEGT)],
                ssems[b],
            )
            cp.start()
            descs.append(cp)
        return descs

    gd = [start_gather(k) for k in range(NT)]
    pend = []
    for k in range(NT):
        gd[k].wait()
        pend.extend(start_scatter(k))
    for cp in pend:
        cp.wait()


def kernel(inputs, categorical_idx):
    outT = _sc_rowgather(inputs.T, categorical_idx)
    return outT.T
